# Initial kernel scaffold; baseline (speedup 1.0000x reference)
import jax
import jax.numpy as jnp
from jax.experimental import pallas as pl


def kernel(table, attention_mask, table_labels_S, table_labels_E, biaffine_edge_S, biaffine_edge_E, W_S, b_S, W_E, b_E):
    raise NotImplementedError("write your pallas kernel here")


# speedup vs baseline: 1.9487x; 1.9487x over previous
import functools

import jax
import jax.numpy as jnp
from jax.experimental import pallas as pl
from jax.experimental.pallas import tpu as pltpu

_Z = 0.3


def _body(CI, B, L, D, am_ref, w_ref, b2_ref, table_ref, biaS_ref, biaE_ref,
          labS_ref, labE_ref, outS_ref, outE_ref, lossS_ref, lossE_ref,
          key_ref, acc_ref):
    b = pl.program_id(0)
    j = pl.program_id(1)
    NJ = L // CI

    @pl.when((b == 0) & (j == 0))
    def _init():
        acc_ref[0] = 0.0
        acc_ref[1] = 0.0

    tbl = table_ref[0].reshape(CI * L, D)
    logits2 = jax.lax.dot_general(
        tbl, w_ref[...], (((1,), (0,)), ((), ())),
        preferred_element_type=jnp.float32,
        precision=jax.lax.Precision.DEFAULT) + b2_ref[...]
    lS = logits2[:, 0].reshape(CI, L) * (1.0 + biaS_ref[0, :, :, 0])
    lE = logits2[:, 1].reshape(CI, L) * (1.0 + biaE_ref[0, :, :, 0])

    yS = labS_ref[0].astype(jnp.float32)
    yE = labE_ref[0].astype(jnp.float32)
    wtS = (labS_ref[0] >= 0).astype(jnp.float32)
    wtE = (labE_ref[0] >= 0).astype(jnp.float32)
    perS = jnp.maximum(lS, 0.0) - lS * yS + jnp.log(1.0 + jnp.exp(-jnp.abs(lS)))
    perE = jnp.maximum(lE, 0.0) - lE * yE + jnp.log(1.0 + jnp.exp(-jnp.abs(lE)))
    acc_ref[0] += jnp.sum(wtS * perS)
    acc_ref[1] += jnp.sum(wtE * perE)

    predS = wtS / (1.0 + jnp.exp(-lS))
    predE = wtE / (1.0 + jnp.exp(-lE))
    keyS = jax.lax.bitcast_convert_type(predS, jnp.int32)
    keyE = jax.lax.bitcast_convert_type(predE, jnp.int32)
    key_ref[pl.ds(b, 1), pl.ds(j * CI, CI), :] = keyS[None]
    key_ref[pl.ds(B + b, 1), pl.ds(j * CI, CI), :] = keyE[None]

    @pl.when((b == B - 1) & (j == NJ - 1))
    def _finish():
        m4 = jnp.sum(jnp.sum(am_ref[...], axis=2), axis=1) - 2
        len4 = jnp.maximum((m4.astype(jnp.float32) * _Z).astype(jnp.int32), 5)
        len4 = jnp.minimum(len4, m4 * m4)
        k8 = jnp.concatenate([len4, len4], axis=0)

        def step(_, lohi):
            lo, hi = lohi
            mid = lo + (hi - lo) // 2
            t = mid.reshape(2 * B, 1, 1)
            ge_cnt = jnp.sum(
                jnp.sum((key_ref[...] >= t).astype(jnp.int32), axis=2), axis=1)
            take = ge_cnt >= k8
            return (jnp.where(take, mid, lo), jnp.where(take, hi, mid))

        lo0 = jnp.zeros((2 * B,), jnp.int32)
        hi0 = jnp.full((2 * B,), 0x7F800000, jnp.int32)
        lo, _hi = jax.lax.fori_loop(0, 31, step, (lo0, hi0))
        msk = (key_ref[...] >= lo.reshape(2 * B, 1, 1)).astype(jnp.float32)
        outS_ref[...] = msk[0:B]
        outE_ref[...] = msk[B:2 * B]
        scale = 1.0 / (B * L * L)
        lossS_ref[...] = jnp.broadcast_to(acc_ref[0] * scale, (1, 1))
        lossE_ref[...] = jnp.broadcast_to(acc_ref[1] * scale, (1, 1))


def kernel(table, attention_mask, table_labels_S, table_labels_E,
           biaffine_edge_S, biaffine_edge_E, W_S, b_S, W_E, b_E):
    B, L, _, D = table.shape
    CI = 16
    NJ = L // CI
    am3 = attention_mask.reshape(B, 1, L)
    w2 = jnp.concatenate([W_S, W_E], axis=1)
    b2 = jnp.concatenate([b_S, b_E], axis=0)[None, :]

    outS, outE, lossS, lossE = pl.pallas_call(
        functools.partial(_body, CI, B, L, D),
        grid=(B, NJ),
        in_specs=[
            pl.BlockSpec((B, 1, L), lambda b, j: (0, 0, 0)),
            pl.BlockSpec((D, 2), lambda b, j: (0, 0)),
            pl.BlockSpec((1, 2), lambda b, j: (0, 0)),
            pl.BlockSpec((1, CI, L, D), lambda b, j: (b, j, 0, 0)),
            pl.BlockSpec((1, CI, L, 1), lambda b, j: (b, j, 0, 0)),
            pl.BlockSpec((1, CI, L, 1), lambda b, j: (b, j, 0, 0)),
            pl.BlockSpec((1, CI, L), lambda b, j: (b, j, 0)),
            pl.BlockSpec((1, CI, L), lambda b, j: (b, j, 0)),
        ],
        out_specs=[
            pl.BlockSpec((B, L, L), lambda b, j: (0, 0, 0)),
            pl.BlockSpec((B, L, L), lambda b, j: (0, 0, 0)),
            pl.BlockSpec((1, 1), lambda b, j: (0, 0)),
            pl.BlockSpec((1, 1), lambda b, j: (0, 0)),
        ],
        out_shape=[
            jax.ShapeDtypeStruct((B, L, L), jnp.float32),
            jax.ShapeDtypeStruct((B, L, L), jnp.float32),
            jax.ShapeDtypeStruct((1, 1), jnp.float32),
            jax.ShapeDtypeStruct((1, 1), jnp.float32),
        ],
        scratch_shapes=[
            pltpu.VMEM((2 * B, L, L), jnp.int32),
            pltpu.SMEM((2,), jnp.float32),
        ],
    )(am3, w2, b2, table, biaffine_edge_S, biaffine_edge_E,
      table_labels_S, table_labels_E)

    return (lossS[0, 0], lossE[0, 0],
            outS.astype(jnp.bool_), outE.astype(jnp.bool_),
            table_labels_S, table_labels_E)


# speedup vs baseline: 2.0229x; 1.0381x over previous
import functools

import jax
import jax.numpy as jnp
from jax.experimental import pallas as pl
from jax.experimental.pallas import tpu as pltpu

_Z = 0.3


def _body(CI, B, L, D, am_ref, w_ref, b2_ref, table_ref, biaS_ref, biaE_ref,
          labS_ref, labE_ref, outS_ref, outE_ref, lossS_ref, lossE_ref,
          key_ref, acc_ref):
    b = pl.program_id(0)
    j = pl.program_id(1)
    NJ = L // CI

    @pl.when((b == 0) & (j == 0))
    def _init():
        acc_ref[0] = 0.0
        acc_ref[1] = 0.0

    tbl = table_ref[0].reshape(CI * L, D)
    logits2 = jax.lax.dot_general(
        tbl, w_ref[...], (((1,), (0,)), ((), ())),
        preferred_element_type=jnp.float32,
        precision=jax.lax.Precision.DEFAULT) + b2_ref[...]
    lS = logits2[:, 0].reshape(CI, L) * (1.0 + biaS_ref[0, :, :, 0])
    lE = logits2[:, 1].reshape(CI, L) * (1.0 + biaE_ref[0, :, :, 0])

    yS = labS_ref[0].astype(jnp.float32)
    yE = labE_ref[0].astype(jnp.float32)
    wtS = (labS_ref[0] >= 0).astype(jnp.float32)
    wtE = (labE_ref[0] >= 0).astype(jnp.float32)
    perS = jnp.maximum(lS, 0.0) - lS * yS + jnp.log(1.0 + jnp.exp(-jnp.abs(lS)))
    perE = jnp.maximum(lE, 0.0) - lE * yE + jnp.log(1.0 + jnp.exp(-jnp.abs(lE)))
    acc_ref[0] += jnp.sum(wtS * perS)
    acc_ref[1] += jnp.sum(wtE * perE)

    predS = wtS / (1.0 + jnp.exp(-lS))
    predE = wtE / (1.0 + jnp.exp(-lE))
    keyS = jax.lax.bitcast_convert_type(predS, jnp.int32)
    keyE = jax.lax.bitcast_convert_type(predE, jnp.int32)
    key_ref[pl.ds(b, 1), pl.ds(j * CI, CI), :] = keyS[None]
    key_ref[pl.ds(B + b, 1), pl.ds(j * CI, CI), :] = keyE[None]

    @pl.when((b == B - 1) & (j == NJ - 1))
    def _finish():
        m4 = jnp.sum(jnp.sum(am_ref[...], axis=2), axis=1) - 2
        len4 = jnp.maximum((m4.astype(jnp.float32) * _Z).astype(jnp.int32), 5)
        len4 = jnp.minimum(len4, m4 * m4)
        k8 = jnp.concatenate([len4, len4], axis=0)

        def step(_, lohi):
            lo, hi = lohi
            mid = lo + (hi - lo) // 2
            t = mid.reshape(2 * B, 1, 1)
            ge_cnt = jnp.sum(
                jnp.sum((key_ref[...] >= t).astype(jnp.int32), axis=2), axis=1)
            take = ge_cnt >= k8
            return (jnp.where(take, mid, lo), jnp.where(take, hi, mid))

        lo0 = jnp.zeros((2 * B,), jnp.int32)
        hi0 = jnp.full((2 * B,), 0x7F800000, jnp.int32)
        lo, _hi = jax.lax.fori_loop(0, 31, step, (lo0, hi0))
        msk = (key_ref[...] >= lo.reshape(2 * B, 1, 1)).astype(jnp.float32)
        outS_ref[...] = msk[0:B]
        outE_ref[...] = msk[B:2 * B]
        scale = 1.0 / (B * L * L)
        lossS_ref[...] = jnp.broadcast_to(acc_ref[0] * scale, (1, 1))
        lossE_ref[...] = jnp.broadcast_to(acc_ref[1] * scale, (1, 1))


def kernel(table, attention_mask, table_labels_S, table_labels_E,
           biaffine_edge_S, biaffine_edge_E, W_S, b_S, W_E, b_E):
    B, L, _, D = table.shape
    CI = 32
    NJ = L // CI
    am3 = attention_mask.reshape(B, 1, L)
    w2 = jnp.concatenate([W_S, W_E], axis=1)
    b2 = jnp.concatenate([b_S, b_E], axis=0)[None, :]

    outS, outE, lossS, lossE = pl.pallas_call(
        functools.partial(_body, CI, B, L, D),
        grid=(B, NJ),
        in_specs=[
            pl.BlockSpec((B, 1, L), lambda b, j: (0, 0, 0)),
            pl.BlockSpec((D, 2), lambda b, j: (0, 0)),
            pl.BlockSpec((1, 2), lambda b, j: (0, 0)),
            pl.BlockSpec((1, CI, L, D), lambda b, j: (b, j, 0, 0)),
            pl.BlockSpec((1, CI, L, 1), lambda b, j: (b, j, 0, 0)),
            pl.BlockSpec((1, CI, L, 1), lambda b, j: (b, j, 0, 0)),
            pl.BlockSpec((1, CI, L), lambda b, j: (b, j, 0)),
            pl.BlockSpec((1, CI, L), lambda b, j: (b, j, 0)),
        ],
        out_specs=[
            pl.BlockSpec((B, L, L), lambda b, j: (0, 0, 0)),
            pl.BlockSpec((B, L, L), lambda b, j: (0, 0, 0)),
            pl.BlockSpec((1, 1), lambda b, j: (0, 0)),
            pl.BlockSpec((1, 1), lambda b, j: (0, 0)),
        ],
        out_shape=[
            jax.ShapeDtypeStruct((B, L, L), jnp.float32),
            jax.ShapeDtypeStruct((B, L, L), jnp.float32),
            jax.ShapeDtypeStruct((1, 1), jnp.float32),
            jax.ShapeDtypeStruct((1, 1), jnp.float32),
        ],
        scratch_shapes=[
            pltpu.VMEM((2 * B, L, L), jnp.int32),
            pltpu.SMEM((2,), jnp.float32),
        ],
    )(am3, w2, b2, table, biaffine_edge_S, biaffine_edge_E,
      table_labels_S, table_labels_E)

    return (lossS[0, 0], lossE[0, 0],
            outS.astype(jnp.bool_), outE.astype(jnp.bool_),
            table_labels_S, table_labels_E)


# speedup vs baseline: 2.0298x; 1.0034x over previous
import functools

import jax
import jax.numpy as jnp
from jax.experimental import pallas as pl
from jax.experimental.pallas import tpu as pltpu

_Z = 0.3


def _body(CI, B, L, D, am_ref, w_ref, b2_ref, table_ref, biaS_ref, biaE_ref,
          labS_ref, labE_ref, outS_ref, outE_ref, lossS_ref, lossE_ref,
          key_ref, acc_ref):
    b = pl.program_id(0)
    j = pl.program_id(1)
    NJ = L // CI

    @pl.when((b == 0) & (j == 0))
    def _init():
        acc_ref[0] = 0.0
        acc_ref[1] = 0.0

    tbl = table_ref[0].reshape(CI * L, D)
    logits2 = jax.lax.dot_general(
        tbl, w_ref[...], (((1,), (0,)), ((), ())),
        preferred_element_type=jnp.float32,
        precision=jax.lax.Precision.DEFAULT) + b2_ref[...]
    lS = logits2[:, 0].reshape(CI, L) * (1.0 + biaS_ref[0, :, :, 0])
    lE = logits2[:, 1].reshape(CI, L) * (1.0 + biaE_ref[0, :, :, 0])

    yS = labS_ref[0].astype(jnp.float32)
    yE = labE_ref[0].astype(jnp.float32)
    wtS = (labS_ref[0] >= 0).astype(jnp.float32)
    wtE = (labE_ref[0] >= 0).astype(jnp.float32)
    eS = jnp.exp(-jnp.abs(lS))
    eE = jnp.exp(-jnp.abs(lE))
    perS = jnp.maximum(lS, 0.0) - lS * yS + jnp.log(1.0 + eS)
    perE = jnp.maximum(lE, 0.0) - lE * yE + jnp.log(1.0 + eE)
    acc_ref[0] += jnp.sum(wtS * perS)
    acc_ref[1] += jnp.sum(wtE * perE)

    def _key(l, wt):
        bits = jax.lax.bitcast_convert_type(l, jnp.int32)
        neg = jnp.bitwise_xor(-1 - bits, jnp.int32(-2147483648))
        k = jnp.where(bits >= 0, bits, neg)
        return jnp.where(wt > 0.0, k, jnp.int32(-2147483648))

    keyS = _key(lS, wtS)
    keyE = _key(lE, wtE)
    key_ref[pl.ds(b, 1), pl.ds(j * CI, CI), :] = keyS[None]
    key_ref[pl.ds(B + b, 1), pl.ds(j * CI, CI), :] = keyE[None]

    @pl.when((b == B - 1) & (j == NJ - 1))
    def _finish():
        m4 = jnp.sum(jnp.sum(am_ref[...], axis=2), axis=1) - 2
        len4 = jnp.maximum((m4.astype(jnp.float32) * _Z).astype(jnp.int32), 5)
        len4 = jnp.minimum(len4, m4 * m4)
        k8 = jnp.concatenate([len4, len4], axis=0)

        def step(_, lohi):
            lo, hi = lohi
            mid = (lo >> 1) + (hi >> 1) + (lo & hi & 1)
            t = mid.reshape(2 * B, 1, 1)
            ge_cnt = jnp.sum(
                jnp.sum((key_ref[...] >= t).astype(jnp.int32), axis=2), axis=1)
            take = ge_cnt >= k8
            return (jnp.where(take, mid, lo), jnp.where(take, hi, mid))

        lo0 = jnp.full((2 * B,), -2147483648, jnp.int32)
        hi0 = jnp.full((2 * B,), 0x7F800000, jnp.int32)
        lo, _hi = jax.lax.fori_loop(0, 32, step, (lo0, hi0))
        msk = (key_ref[...] >= lo.reshape(2 * B, 1, 1)).astype(jnp.float32)
        outS_ref[...] = msk[0:B]
        outE_ref[...] = msk[B:2 * B]
        scale = 1.0 / (B * L * L)
        lossS_ref[...] = jnp.broadcast_to(acc_ref[0] * scale, (1, 1))
        lossE_ref[...] = jnp.broadcast_to(acc_ref[1] * scale, (1, 1))


def kernel(table, attention_mask, table_labels_S, table_labels_E,
           biaffine_edge_S, biaffine_edge_E, W_S, b_S, W_E, b_E):
    B, L, _, D = table.shape
    CI = 32
    NJ = L // CI
    am3 = attention_mask.reshape(B, 1, L)
    w2 = jnp.concatenate([W_S, W_E], axis=1)
    b2 = jnp.concatenate([b_S, b_E], axis=0)[None, :]

    outS, outE, lossS, lossE = pl.pallas_call(
        functools.partial(_body, CI, B, L, D),
        grid=(B, NJ),
        in_specs=[
            pl.BlockSpec((B, 1, L), lambda b, j: (0, 0, 0)),
            pl.BlockSpec((D, 2), lambda b, j: (0, 0)),
            pl.BlockSpec((1, 2), lambda b, j: (0, 0)),
            pl.BlockSpec((1, CI, L, D), lambda b, j: (b, j, 0, 0)),
            pl.BlockSpec((1, CI, L, 1), lambda b, j: (b, j, 0, 0)),
            pl.BlockSpec((1, CI, L, 1), lambda b, j: (b, j, 0, 0)),
            pl.BlockSpec((1, CI, L), lambda b, j: (b, j, 0)),
            pl.BlockSpec((1, CI, L), lambda b, j: (b, j, 0)),
        ],
        out_specs=[
            pl.BlockSpec((B, L, L), lambda b, j: (0, 0, 0)),
            pl.BlockSpec((B, L, L), lambda b, j: (0, 0, 0)),
            pl.BlockSpec((1, 1), lambda b, j: (0, 0)),
            pl.BlockSpec((1, 1), lambda b, j: (0, 0)),
        ],
        out_shape=[
            jax.ShapeDtypeStruct((B, L, L), jnp.float32),
            jax.ShapeDtypeStruct((B, L, L), jnp.float32),
            jax.ShapeDtypeStruct((1, 1), jnp.float32),
            jax.ShapeDtypeStruct((1, 1), jnp.float32),
        ],
        scratch_shapes=[
            pltpu.VMEM((2 * B, L, L), jnp.int32),
            pltpu.SMEM((2,), jnp.float32),
        ],
    )(am3, w2, b2, table, biaffine_edge_S, biaffine_edge_E,
      table_labels_S, table_labels_E)

    return (lossS[0, 0], lossE[0, 0],
            outS.astype(jnp.bool_), outE.astype(jnp.bool_),
            table_labels_S, table_labels_E)
